# baseline (speedup 1.0000x reference)
import functools

import jax
import jax.numpy as jnp
from jax import lax
from jax.experimental import pallas as pl
from jax.experimental.pallas import tpu as pltpu
from jax.experimental.pallas import tpu_sc as plsc

B = 1024
D = 128
CB = 100000
H = 512
CB_TILE = 2000

_PREC = jax.lax.Precision.DEFAULT


_NTILES = CB // CB_TILE


def _dist_stage(step, codes, qsq, tile, epi_dot, epi_csq, out_dot, out_csq,
                best_val, best_idx, idx_ref):
    m2dot = jax.lax.dot_general(
        codes, tile, (((1,), (1,)), ((), ())),
        preferred_element_type=jnp.float32, precision=_PREC)
    out_dot[...] = m2dot
    out_csq[...] = jnp.sum(tile * tile, axis=1)[None, :]

    j = step - 1
    d2 = (qsq + epi_dot[...]) + epi_csq[...]
    local_min = jnp.min(d2, axis=1, keepdims=True)
    col = jax.lax.broadcasted_iota(jnp.int32, d2.shape, 1)
    local_arg = jnp.min(
        jnp.where(d2 == local_min, col, CB_TILE),
        axis=1, keepdims=True) + j * CB_TILE
    bv = best_val[...]
    take = (j == 0) | ((j > 0) & (local_min < bv))
    best_val[...] = jnp.where(take, local_min, bv)
    nbi = jnp.where(take, local_arg, best_idx[...])
    best_idx[...] = nbi
    idx_ref[...] = nbi


def _dist_argmin_body(codes_ref, cb_ref, idx_ref, dot_a, dot_b, csq_a, csq_b,
                      best_val, best_idx):
    step = pl.program_id(0)
    codes = codes_ref[...]
    qsq = 0.25 * jnp.sum(codes * codes, axis=1, keepdims=True)
    tile = cb_ref[...]

    @pl.when(jax.lax.rem(step, 2) == 0)
    def _():
        _dist_stage(step, codes, qsq, tile, dot_b, csq_b, dot_a, csq_a,
                    best_val, best_idx, idx_ref)

    @pl.when(jax.lax.rem(step, 2) == 1)
    def _():
        _dist_stage(step, codes, qsq, tile, dot_a, csq_a, dot_b, csq_b,
                    best_val, best_idx, idx_ref)


def _nearest_idx(codes, codebook):
    idx = pl.pallas_call(
        _dist_argmin_body,
        grid=(_NTILES + 1,),
        in_specs=[
            pl.BlockSpec((B, D), lambda i: (0, 0)),
            pl.BlockSpec((CB_TILE, D), lambda i: (jnp.minimum(i, _NTILES - 1), 0)),
        ],
        out_specs=pl.BlockSpec((B, 1), lambda i: (0, 0)),
        out_shape=jax.ShapeDtypeStruct((B, 1), jnp.int32),
        scratch_shapes=[
            pltpu.VMEM((B, CB_TILE), jnp.float32),
            pltpu.VMEM((B, CB_TILE), jnp.float32),
            pltpu.VMEM((1, CB_TILE), jnp.float32),
            pltpu.VMEM((1, CB_TILE), jnp.float32),
            pltpu.VMEM((B, 1), jnp.float32),
            pltpu.VMEM((B, 1), jnp.int32),
        ],
    )(codes * -2.0, codebook)
    return idx.reshape(B)


_NW = 32
_BPW = B // _NW


@functools.cache
def _sc_gather_kernel():
    def body(table_hbm, idx_hbm, out_hbm, idx_v, rows_v, sem):
        wid = lax.axis_index("s") * 2 + lax.axis_index("c")
        base = wid * _BPW
        pltpu.sync_copy(idx_hbm.at[pl.ds(base, _BPW)], idx_v)
        pltpu.async_copy(table_hbm.at[idx_v], rows_v, sem).wait()
        pltpu.sync_copy(rows_v, out_hbm.at[pl.ds(base, _BPW)])

    return pl.kernel(
        body,
        mesh=plsc.VectorSubcoreMesh(core_axis_name="c", subcore_axis_name="s"),
        out_type=jax.ShapeDtypeStruct((B, D), jnp.float32),
        scratch_types=[
            pltpu.VMEM((_BPW,), jnp.int32),
            pltpu.VMEM((_BPW, D), jnp.float32),
            pltpu.SemaphoreType.DMA,
        ],
    )


def _mlp_body(prev_ref, w1_ref, b1_ref, wu_ref, bu_ref, ws_ref, bs_ref,
              mu_ref, ls_ref):
    prev = prev_ref[...]
    h = jax.lax.dot_general(
        prev, w1_ref[...], (((1,), (1,)), ((), ())),
        preferred_element_type=jnp.float32, precision=_PREC)
    h = jnp.maximum(h + b1_ref[...], 0.0)
    mu_ref[...] = jax.lax.dot_general(
        h, wu_ref[...], (((1,), (1,)), ((), ())),
        preferred_element_type=jnp.float32, precision=_PREC) + bu_ref[...]
    ls_ref[...] = jax.lax.dot_general(
        h, ws_ref[...], (((1,), (1,)), ((), ())),
        preferred_element_type=jnp.float32, precision=_PREC) + bs_ref[...]


def _mlp(prev, W1, b1, Wu, bu, Ws, bs):
    return pl.pallas_call(
        _mlp_body,
        out_shape=(
            jax.ShapeDtypeStruct((B, D), jnp.float32),
            jax.ShapeDtypeStruct((B, D), jnp.float32),
        ),
    )(prev, W1, b1.reshape(1, H), Wu, bu.reshape(1, D), Ws, bs.reshape(1, D))


def kernel(codes, codebook, W1, b1, Wu, bu, Ws, bs):
    chosen = _nearest_idx(codes, codebook)
    prev = _sc_gather_kernel()(codebook, chosen)
    return _mlp(prev, W1, b1, Wu, bu, Ws, bs)


# speedup vs baseline: 1.2180x; 1.2180x over previous
import functools

import jax
import jax.numpy as jnp
from jax import lax
from jax.experimental import pallas as pl
from jax.experimental.pallas import tpu as pltpu
from jax.experimental.pallas import tpu_sc as plsc

B = 1024
D = 128
CB = 100000
H = 512
CB_TILE = 2000

_PREC = jax.lax.Precision.DEFAULT


_NTILES = CB // CB_TILE


def _dist_argmin_body(codes_ref, cb_ref, idx_ref, best_val, best_idx):
    step = pl.program_id(0)
    codes = codes_ref[...]
    tile = cb_ref[...]
    m2dot = jax.lax.dot_general(
        codes, tile, (((1,), (1,)), ((), ())),
        preferred_element_type=jnp.float32, precision=_PREC)
    qsq = 0.25 * jnp.sum(codes * codes, axis=1, keepdims=True)
    csq = jnp.sum(tile * tile, axis=1)
    d2 = (qsq + m2dot) + csq[None, :]
    local_min = jnp.min(d2, axis=1, keepdims=True)
    col = jax.lax.broadcasted_iota(jnp.int32, d2.shape, 1)
    local_arg = jnp.min(
        jnp.where(d2 == local_min, col, CB_TILE),
        axis=1, keepdims=True) + step * CB_TILE

    @pl.when(step == 0)
    def _():
        best_val[...] = local_min
        best_idx[...] = local_arg

    @pl.when(step > 0)
    def _():
        bv = best_val[...]
        take = local_min < bv
        best_val[...] = jnp.where(take, local_min, bv)
        best_idx[...] = jnp.where(take, local_arg, best_idx[...])

    @pl.when(step == pl.num_programs(0) - 1)
    def _():
        idx_ref[...] = best_idx[...]


def _nearest_idx(codes, codebook):
    idx = pl.pallas_call(
        _dist_argmin_body,
        grid=(_NTILES,),
        in_specs=[
            pl.BlockSpec((B, D), lambda i: (0, 0)),
            pl.BlockSpec((CB_TILE, D), lambda i: (i, 0)),
        ],
        out_specs=pl.BlockSpec((B, 1), lambda i: (0, 0)),
        out_shape=jax.ShapeDtypeStruct((B, 1), jnp.int32),
        scratch_shapes=[
            pltpu.VMEM((B, 1), jnp.float32),
            pltpu.VMEM((B, 1), jnp.int32),
        ],
    )(codes * -2.0, codebook)
    return idx.reshape(B)


_NW = 32
_BPW = B // _NW


@functools.cache
def _sc_gather_kernel():
    def body(table_hbm, idx_hbm, out_hbm, idx_v, rows_v, sem):
        wid = lax.axis_index("s") * 2 + lax.axis_index("c")
        base = wid * _BPW
        pltpu.sync_copy(idx_hbm.at[pl.ds(base, _BPW)], idx_v)
        pltpu.async_copy(table_hbm.at[idx_v], rows_v, sem).wait()
        pltpu.sync_copy(rows_v, out_hbm.at[pl.ds(base, _BPW)])

    return pl.kernel(
        body,
        mesh=plsc.VectorSubcoreMesh(core_axis_name="c", subcore_axis_name="s"),
        out_type=jax.ShapeDtypeStruct((B, D), jnp.float32),
        scratch_types=[
            pltpu.VMEM((_BPW,), jnp.int32),
            pltpu.VMEM((_BPW, D), jnp.float32),
            pltpu.SemaphoreType.DMA,
        ],
    )


def _mlp_body(prev_ref, w1_ref, b1_ref, wu_ref, bu_ref, ws_ref, bs_ref,
              mu_ref, ls_ref):
    prev = prev_ref[...]
    h = jax.lax.dot_general(
        prev, w1_ref[...], (((1,), (1,)), ((), ())),
        preferred_element_type=jnp.float32, precision=_PREC)
    h = jnp.maximum(h + b1_ref[...], 0.0)
    mu_ref[...] = jax.lax.dot_general(
        h, wu_ref[...], (((1,), (1,)), ((), ())),
        preferred_element_type=jnp.float32, precision=_PREC) + bu_ref[...]
    ls_ref[...] = jax.lax.dot_general(
        h, ws_ref[...], (((1,), (1,)), ((), ())),
        preferred_element_type=jnp.float32, precision=_PREC) + bs_ref[...]


def _mlp(prev, W1, b1, Wu, bu, Ws, bs):
    return pl.pallas_call(
        _mlp_body,
        out_shape=(
            jax.ShapeDtypeStruct((B, D), jnp.float32),
            jax.ShapeDtypeStruct((B, D), jnp.float32),
        ),
    )(prev, W1, b1.reshape(1, H), Wu, bu.reshape(1, D), Ws, bs.reshape(1, D))


def kernel(codes, codebook, W1, b1, Wu, bu, Ws, bs):
    chosen = _nearest_idx(codes, codebook)
    prev = _sc_gather_kernel()(codebook, chosen)
    return _mlp(prev, W1, b1, Wu, bu, Ws, bs)


# speedup vs baseline: 1.3825x; 1.1351x over previous
import functools

import jax
import jax.numpy as jnp
from jax import lax
from jax.experimental import pallas as pl
from jax.experimental.pallas import tpu as pltpu
from jax.experimental.pallas import tpu_sc as plsc

B = 1024
D = 128
CB = 100000
H = 512
CB_TILE = 5000

_PREC = jax.lax.Precision.DEFAULT


_NTILES = CB // CB_TILE
_NSUB = 4
_SUB = CB_TILE // _NSUB


def _dist_argmin_body(codes_ref, cb_ref, idx_ref, best_val, best_idx):
    step = pl.program_id(0)
    codes = codes_ref[...]
    tile = cb_ref[...]
    m2dot = jax.lax.dot_general(
        codes, tile, (((1,), (1,)), ((), ())),
        preferred_element_type=jnp.float32, precision=_PREC)
    qsq = 0.25 * jnp.sum(codes * codes, axis=1, keepdims=True)
    csq = jnp.sum(tile * tile, axis=1)
    d2 = (qsq + m2dot) + csq[None, :]
    local_min = jnp.min(d2, axis=1, keepdims=True)
    col = jax.lax.broadcasted_iota(jnp.int32, d2.shape, 1)
    local_arg = jnp.min(
        jnp.where(d2 == local_min, col, CB_TILE),
        axis=1, keepdims=True) + step * CB_TILE

    @pl.when(step == 0)
    def _():
        best_val[...] = local_min
        best_idx[...] = local_arg

    @pl.when(step > 0)
    def _():
        bv = best_val[...]
        take = local_min < bv
        best_val[...] = jnp.where(take, local_min, bv)
        best_idx[...] = jnp.where(take, local_arg, best_idx[...])

    @pl.when(step == pl.num_programs(0) - 1)
    def _():
        idx_ref[...] = best_idx[...]


def _nearest_idx(codes, codebook):
    idx = pl.pallas_call(
        _dist_argmin_body,
        grid=(_NTILES,),
        in_specs=[
            pl.BlockSpec((B, D), lambda i: (0, 0)),
            pl.BlockSpec((CB_TILE, D), lambda i: (i, 0)),
        ],
        out_specs=pl.BlockSpec((B, 1), lambda i: (0, 0)),
        out_shape=jax.ShapeDtypeStruct((B, 1), jnp.int32),
        scratch_shapes=[
            pltpu.VMEM((B, 1), jnp.float32),
            pltpu.VMEM((B, 1), jnp.int32),
        ],
    )(codes * -2.0, codebook)
    return idx.reshape(B)


_NW = 32
_BPW = B // _NW


@functools.cache
def _sc_gather_kernel():
    def body(table_hbm, idx_hbm, out_hbm, idx_v, rows_v, sem):
        wid = lax.axis_index("s") * 2 + lax.axis_index("c")
        base = wid * _BPW
        pltpu.sync_copy(idx_hbm.at[pl.ds(base, _BPW)], idx_v)
        pltpu.async_copy(table_hbm.at[idx_v], rows_v, sem).wait()
        pltpu.sync_copy(rows_v, out_hbm.at[pl.ds(base, _BPW)])

    return pl.kernel(
        body,
        mesh=plsc.VectorSubcoreMesh(core_axis_name="c", subcore_axis_name="s"),
        out_type=jax.ShapeDtypeStruct((B, D), jnp.float32),
        scratch_types=[
            pltpu.VMEM((_BPW,), jnp.int32),
            pltpu.VMEM((_BPW, D), jnp.float32),
            pltpu.SemaphoreType.DMA,
        ],
    )


def _mlp_body(prev_ref, w1_ref, b1_ref, wu_ref, bu_ref, ws_ref, bs_ref,
              mu_ref, ls_ref):
    prev = prev_ref[...]
    h = jax.lax.dot_general(
        prev, w1_ref[...], (((1,), (1,)), ((), ())),
        preferred_element_type=jnp.float32, precision=_PREC)
    h = jnp.maximum(h + b1_ref[...], 0.0)
    mu_ref[...] = jax.lax.dot_general(
        h, wu_ref[...], (((1,), (1,)), ((), ())),
        preferred_element_type=jnp.float32, precision=_PREC) + bu_ref[...]
    ls_ref[...] = jax.lax.dot_general(
        h, ws_ref[...], (((1,), (1,)), ((), ())),
        preferred_element_type=jnp.float32, precision=_PREC) + bs_ref[...]


def _mlp(prev, W1, b1, Wu, bu, Ws, bs):
    return pl.pallas_call(
        _mlp_body,
        out_shape=(
            jax.ShapeDtypeStruct((B, D), jnp.float32),
            jax.ShapeDtypeStruct((B, D), jnp.float32),
        ),
    )(prev, W1, b1.reshape(1, H), Wu, bu.reshape(1, D), Ws, bs.reshape(1, D))


def kernel(codes, codebook, W1, b1, Wu, bu, Ws, bs):
    chosen = _nearest_idx(codes, codebook)
    prev = _sc_gather_kernel()(codebook, chosen)
    return _mlp(prev, W1, b1, Wu, bu, Ws, bs)
